# Initial kernel scaffold; baseline (speedup 1.0000x reference)
import jax
import jax.numpy as jnp
from jax.experimental import pallas as pl


def kernel(f, s, edge_index, edge_f, edge_s, net, self_net):
    raise NotImplementedError("write your pallas kernel here")


# speedup vs baseline: 8.4991x; 8.4991x over previous
import functools

import jax
import jax.numpy as jnp
from jax import lax
from jax.experimental import pallas as pl
from jax.experimental.pallas import tpu as pltpu
from jax.experimental.pallas import tpu_sc as plsc

N = 50000
E = 800000
FD = 2
SD = 32
HD = 32
PSTEP = 4

NP_ = 50176
EP_ = 819200
TW = 48
MF = 16
CHUNK = 128
NWORK = 32
CPW = EP_ // (NWORK * CHUNK)
STRIPE = NP_ // 16

BE = 2048
BN = 1024

@functools.lru_cache(maxsize=None)
def _build_gather():
    mesh = plsc.VectorSubcoreMesh(core_axis_name="c", subcore_axis_name="s")

    @functools.partial(
        pl.kernel,
        out_type=(
            jax.ShapeDtypeStruct((EP_, TW), jnp.float32),
            jax.ShapeDtypeStruct((EP_, TW), jnp.float32),
        ),
        scratch_types=[
            pltpu.VMEM((CHUNK,), jnp.int32),
            pltpu.VMEM((CHUNK, TW), jnp.float32),
            pltpu.SemaphoreType.DMA,
        ],
        mesh=mesh,
        compiler_params=pltpu.CompilerParams(use_tc_tiling_on_sc=False),
    )
    def _gather_k(tab, srcp, dstp, gsrc, gdst, idxbuf, rowbuf, sem):
        wid = lax.axis_index("s") * 2 + lax.axis_index("c")

        def chunk(t, _):
            base = (wid * CPW + t) * CHUNK
            pltpu.sync_copy(srcp.at[pl.ds(base, CHUNK)], idxbuf)
            pltpu.async_copy(tab.at[idxbuf], rowbuf, sem).wait()
            pltpu.sync_copy(rowbuf, gsrc.at[pl.ds(base, CHUNK)])
            pltpu.sync_copy(dstp.at[pl.ds(base, CHUNK)], idxbuf)
            pltpu.async_copy(tab.at[idxbuf], rowbuf, sem).wait()
            pltpu.sync_copy(rowbuf, gdst.at[pl.ds(base, CHUNK)])
            return _

        lax.fori_loop(0, CPW, chunk, None)

    return _gather_k


@functools.lru_cache(maxsize=None)
def _build_scatter(w):
    mesh = plsc.VectorSubcoreMesh(core_axis_name="c", subcore_axis_name="s")

    @functools.partial(
        pl.kernel,
        out_type=jax.ShapeDtypeStruct((2 * NP_, w), jnp.float32),
        scratch_types=[
            pltpu.VMEM((CHUNK,), jnp.int32),
            pltpu.VMEM((CHUNK, w), jnp.float32),
            pltpu.VMEM_SHARED((NP_, w), jnp.float32),
            pltpu.SemaphoreType.DMA,
        ],
        mesh=mesh,
        compiler_params=pltpu.CompilerParams(use_tc_tiling_on_sc=False),
    )
    def _scatter_k(msg, idx, zrows, part, idxbuf, rowbuf, accum, sem):
        c = lax.axis_index("c")
        s_ = lax.axis_index("s")
        wid = s_ * 2 + c
        sbase = s_ * STRIPE
        pltpu.sync_copy(zrows.at[pl.ds(sbase, STRIPE)],
                        accum.at[pl.ds(sbase, STRIPE)])
        plsc.subcore_barrier()

        def chunk(t, _):
            base = (wid * CPW + t) * CHUNK
            pltpu.sync_copy(idx.at[pl.ds(base, CHUNK)], idxbuf)
            pltpu.sync_copy(msg.at[pl.ds(base, CHUNK)], rowbuf)
            pltpu.sync_copy(rowbuf, accum.at[idxbuf], add=True)
            return _

        lax.fori_loop(0, CPW, chunk, None)
        plsc.subcore_barrier()
        pltpu.sync_copy(accum.at[pl.ds(sbase, STRIPE)],
                        part.at[pl.ds(c * NP_ + sbase, STRIPE)])

    return _scatter_k


def _silu(x):
    return x / (1.0 + jnp.exp(-x))


def _edge_body(gs_ref, gd_ref, ef_ref, es_ref,
               w1, b1, w2, b2, w3, b3, msgf_ref, msgs_ref):
    gs = gs_ref[...]
    gd = gd_ref[...]
    ef = ef_ref[...]
    fj = [jnp.concatenate(
        [gs[:, 2 * j:2 * j + 2], gd[:, 2 * j:2 * j + 2], ef[:, j:j + 1]],
        axis=1) for j in range(3)]
    gcols = []
    for i in range(5):
        gi = (fj[0][:, i:i + 1] * fj[0]
              + fj[1][:, i:i + 1] * fj[1]
              + fj[2][:, i:i + 1] * fj[2])
        gcols.append(gi)
    gram = jnp.concatenate(gcols, axis=1)
    nrm = jnp.sqrt(jnp.sum(gram * gram, axis=1, keepdims=True))
    gram = gram / jnp.maximum(nrm, 1e-12)
    x = jnp.concatenate([gram, gs[:, 16:48], gd[:, 16:48], es_ref[...]],
                        axis=1)
    h = _silu(jnp.dot(x, w1[...], preferred_element_type=jnp.float32)
              + b1[...])
    h = _silu(jnp.dot(h, w2[...], preferred_element_type=jnp.float32)
              + b2[...])
    cc = (jnp.dot(h, w3[...], preferred_element_type=jnp.float32)
          + b3[...])
    msgs_ref[...] = cc[:, 10:42]
    mcols = []
    for i in range(3):
        for k in range(2):
            m = (fj[i][:, 0:1] * cc[:, k:k + 1]
                 + fj[i][:, 1:2] * cc[:, 2 + k:3 + k]
                 + fj[i][:, 2:3] * cc[:, 4 + k:5 + k]
                 + fj[i][:, 3:4] * cc[:, 6 + k:7 + k]
                 + fj[i][:, 4:5] * cc[:, 8 + k:9 + k])
            mcols.append(m)
    one = jnp.ones_like(mcols[0])
    zer = jnp.zeros((mcols[0].shape[0], MF - 7), jnp.float32)
    msgf_ref[...] = jnp.concatenate(mcols + [one, zer], axis=1)


def _edge_call(gsrc, gdst, efp, esp, nw):
    nb = EP_ // BE
    full = lambda a: pl.BlockSpec(a.shape, lambda i: (0,) * a.ndim)
    return pl.pallas_call(
        _edge_body,
        grid=(nb,),
        in_specs=[
            pl.BlockSpec((BE, TW), lambda i: (i, 0)),
            pl.BlockSpec((BE, TW), lambda i: (i, 0)),
            pl.BlockSpec((BE, 3), lambda i: (i, 0)),
            pl.BlockSpec((BE, 4), lambda i: (i, 0)),
        ] + [full(a) for a in nw],
        out_specs=[
            pl.BlockSpec((BE, MF), lambda i: (i, 0)),
            pl.BlockSpec((BE, SD), lambda i: (i, 0)),
        ],
        out_shape=[
            jax.ShapeDtypeStruct((EP_, MF), jnp.float32),
            jax.ShapeDtypeStruct((EP_, SD), jnp.float32),
        ],
    )(gsrc, gdst, efp, esp, *nw)


def _node_body(tab_ref, fp0, fp1, sp0, sp1,
               w1, b1, w2, b2, w3, b3, out_ref):
    tab = tab_ref[...]
    fsum = fp0[...] + fp1[...]
    inv = 1.0 / jnp.maximum(fsum[:, 6:7], 1.0)
    ssum = (sp0[...] + sp1[...]) * inv
    tj = [jnp.concatenate(
        [tab[:, 2 * j:2 * j + 2], fsum[:, 2 * j:2 * j + 2] * inv], axis=1)
        for j in range(3)]
    gcols = []
    for i in range(4):
        gi = (tj[0][:, i:i + 1] * tj[0]
              + tj[1][:, i:i + 1] * tj[1]
              + tj[2][:, i:i + 1] * tj[2])
        gcols.append(gi)
    gram = jnp.concatenate(gcols, axis=1)
    nrm = jnp.sqrt(jnp.sum(gram * gram, axis=1, keepdims=True))
    gram = gram / jnp.maximum(nrm, 1e-12)
    x = jnp.concatenate([gram, tab[:, 16:48], ssum], axis=1)
    h = _silu(jnp.dot(x, w1[...], preferred_element_type=jnp.float32)
              + b1[...])
    h = _silu(jnp.dot(h, w2[...], preferred_element_type=jnp.float32)
              + b2[...])
    tc = (jnp.dot(h, w3[...], preferred_element_type=jnp.float32)
          + b3[...])
    fcols = []
    for i in range(3):
        for k in range(2):
            m = (tj[i][:, 0:1] * tc[:, k:k + 1]
                 + tj[i][:, 1:2] * tc[:, 2 + k:3 + k]
                 + tj[i][:, 2:3] * tc[:, 4 + k:5 + k]
                 + tj[i][:, 3:4] * tc[:, 6 + k:7 + k])
            fcols.append(m)
    zer = jnp.zeros((tab.shape[0], 10), jnp.float32)
    out_ref[...] = jnp.concatenate(fcols + [zer, tc[:, 8:40]], axis=1)


def _node_call(tab, fpart, spart, sw):
    nb = NP_ // BN
    off = NP_ // BN
    full = lambda a: pl.BlockSpec(a.shape, lambda i: (0,) * a.ndim)
    return pl.pallas_call(
        _node_body,
        grid=(nb,),
        in_specs=[
            pl.BlockSpec((BN, TW), lambda i: (i, 0)),
            pl.BlockSpec((BN, MF), lambda i: (i, 0)),
            pl.BlockSpec((BN, MF), lambda i: (i + off, 0)),
            pl.BlockSpec((BN, SD), lambda i: (i, 0)),
            pl.BlockSpec((BN, SD), lambda i: (i + off, 0)),
        ] + [full(a) for a in sw],
        out_specs=pl.BlockSpec((BN, TW), lambda i: (i, 0)),
        out_shape=jax.ShapeDtypeStruct((NP_, TW), jnp.float32),
    )(tab, fpart, fpart, spart, spart, *sw)


def _weights(p):
    return (p["W1"], p["b1"].reshape(1, -1),
            p["W2"], p["b2"].reshape(1, -1),
            p["W3"], p["b3"].reshape(1, -1))


def kernel(f, s, edge_index, edge_f, edge_s, net, self_net):
    ei = edge_index.astype(jnp.int32)
    pad = jnp.full((EP_ - E,), N, jnp.int32)
    srcp = jnp.concatenate([ei[0], pad])
    dstp = jnp.concatenate([ei[1], pad])
    efp = jnp.pad(edge_f.reshape(E, 3), ((0, EP_ - E), (0, 0)))
    esp = jnp.pad(edge_s, ((0, EP_ - E), (0, 0)))
    tab = jnp.concatenate([
        jnp.pad(f.reshape(N, 6), ((0, NP_ - N), (0, 0))),
        jnp.zeros((NP_, 10), jnp.float32),
        jnp.pad(s, ((0, NP_ - N), (0, 0))),
    ], axis=1)
    zf = jnp.zeros((NP_, MF), jnp.float32)
    zs = jnp.zeros((NP_, SD), jnp.float32)
    nw = _weights(net)
    sw = _weights(self_net)
    gather_k = _build_gather()
    scatter_f = _build_scatter(MF)
    scatter_s = _build_scatter(SD)
    for _ in range(PSTEP):
        gsrc, gdst = gather_k(tab, srcp, dstp)
        msgf, msgs = _edge_call(gsrc, gdst, efp, esp, nw)
        fpart = scatter_f(msgf, srcp, zf)
        spart = scatter_s(msgs, srcp, zs)
        tab = _node_call(tab, fpart, spart, sw)
    return tab[:N, :6].reshape(N, 3, FD), tab[:N, 16:48]


# speedup vs baseline: 18.4791x; 2.1742x over previous
import functools

import jax
import jax.numpy as jnp
import numpy as np
from jax import lax
from jax.experimental import pallas as pl
from jax.experimental.pallas import tpu as pltpu
from jax.experimental.pallas import tpu_sc as plsc

N = 50000
E = 800000
FD = 2
SD = 32
HD = 32
PSTEP = 4

NP_ = 50176
EP_ = 819200
TW = 48
MF = 16
CHUNK = 128
NWORK = 32
CPW = EP_ // (NWORK * CHUNK)
STRIPE = NP_ // 16

BE = 2048
BN = 1024

@functools.lru_cache(maxsize=None)
def _build_gather():
    mesh = plsc.VectorSubcoreMesh(core_axis_name="c", subcore_axis_name="s")

    @functools.partial(
        pl.kernel,
        out_type=(
            jax.ShapeDtypeStruct((EP_, TW), jnp.float32),
            jax.ShapeDtypeStruct((EP_, TW), jnp.float32),
        ),
        scratch_types=[
            pltpu.VMEM((CHUNK,), jnp.int32),
            pltpu.VMEM((CHUNK, TW), jnp.float32),
            pltpu.SemaphoreType.DMA,
        ],
        mesh=mesh,
        compiler_params=pltpu.CompilerParams(use_tc_tiling_on_sc=False),
    )
    def _gather_k(tab, srcp, dstp, gsrc, gdst, idxbuf, rowbuf, sem):
        wid = lax.axis_index("s") * 2 + lax.axis_index("c")

        def chunk(t, _):
            base = (wid * CPW + t) * CHUNK
            pltpu.sync_copy(srcp.at[pl.ds(base, CHUNK)], idxbuf)
            pltpu.async_copy(tab.at[idxbuf], rowbuf, sem).wait()
            pltpu.sync_copy(rowbuf, gsrc.at[pl.ds(base, CHUNK)])
            pltpu.sync_copy(dstp.at[pl.ds(base, CHUNK)], idxbuf)
            pltpu.async_copy(tab.at[idxbuf], rowbuf, sem).wait()
            pltpu.sync_copy(rowbuf, gdst.at[pl.ds(base, CHUNK)])
            return _

        lax.fori_loop(0, CPW, chunk, None)

    return _gather_k


@functools.lru_cache(maxsize=None)
def _build_scatter(w):
    mesh = plsc.VectorSubcoreMesh(core_axis_name="c", subcore_axis_name="s")

    @functools.partial(
        pl.kernel,
        out_type=jax.ShapeDtypeStruct((2 * NP_, w), jnp.float32),
        scratch_types=[
            pltpu.VMEM((CHUNK,), jnp.int32),
            pltpu.VMEM((CHUNK, w), jnp.float32),
            pltpu.VMEM_SHARED((NP_, w), jnp.float32),
            pltpu.SemaphoreType.DMA,
        ],
        mesh=mesh,
        compiler_params=pltpu.CompilerParams(use_tc_tiling_on_sc=False),
    )
    def _scatter_k(msg, idx, zrows, part, idxbuf, rowbuf, accum, sem):
        c = lax.axis_index("c")
        s_ = lax.axis_index("s")
        wid = s_ * 2 + c
        sbase = s_ * STRIPE
        pltpu.sync_copy(zrows.at[pl.ds(sbase, STRIPE)],
                        accum.at[pl.ds(sbase, STRIPE)])
        plsc.subcore_barrier()

        def chunk(t, _):
            base = (wid * CPW + t) * CHUNK
            pltpu.sync_copy(idx.at[pl.ds(base, CHUNK)], idxbuf)
            pltpu.sync_copy(msg.at[pl.ds(base, CHUNK)], rowbuf)
            pltpu.sync_copy(rowbuf, accum.at[idxbuf], add=True)
            return _

        lax.fori_loop(0, CPW, chunk, None)
        plsc.subcore_barrier()
        pltpu.sync_copy(accum.at[pl.ds(sbase, STRIPE)],
                        part.at[pl.ds(c * NP_ + sbase, STRIPE)])

    return _scatter_k


def _fcol(a, q):
    if q < 2:
        return ("gs", 2 * a + q)
    if q < 4:
        return ("gd", 2 * a + q - 2)
    return ("ef", a)


def _tcol(a, q):
    if q < 2:
        return ("tab", 2 * a + q)
    return ("fc", 2 * a + q - 2)


def _sel(shapes, entries):
    mats = {k: np.zeros(v, np.float32) for k, v in shapes.items()}
    for (src, row), col in entries:
        mats[src][row, col] = 1.0
    return mats


def _edge_consts():
    shapes = {"gs": (TW, 75), "gd": (TW, 75), "ef": (3, 75)}
    EA = _sel(shapes, [(_fcol(j, i), j * 25 + i * 5 + k)
                       for j in range(3) for i in range(5) for k in range(5)])
    EB = _sel(shapes, [(_fcol(j, k), j * 25 + i * 5 + k)
                       for j in range(3) for i in range(5) for k in range(5)])
    EC = np.zeros((75, 25), np.float32)
    for j in range(3):
        for i in range(5):
            for k in range(5):
                EC[j * 25 + i * 5 + k, i * 5 + k] = 1.0
    sh2 = {"gs": (TW, 30), "gd": (TW, 30), "ef": (3, 30)}
    MA = _sel(sh2, [(_fcol(i, j), i * 10 + k * 5 + j)
                    for i in range(3) for k in range(2) for j in range(5)])
    MB = np.zeros((42, 30), np.float32)
    MC = np.zeros((30, MF), np.float32)
    for i in range(3):
        for k in range(2):
            for j in range(5):
                MB[2 * j + k, i * 10 + k * 5 + j] = 1.0
                MC[i * 10 + k * 5 + j, i * 2 + k] = 1.0
    MS = np.zeros((42, SD), np.float32)
    for m in range(SD):
        MS[10 + m, m] = 1.0
    CNT = np.zeros((1, MF), np.float32)
    CNT[0, 6] = 1.0
    return (EA["gs"], EA["gd"], EA["ef"], EB["gs"], EB["gd"], EB["ef"], EC,
            MA["gs"], MA["gd"], MA["ef"], MB, MC, MS, CNT)


def _node_consts():
    shapes = {"tab": (TW, 48), "fc": (MF, 48)}
    NA = _sel(shapes, [(_tcol(j, i), j * 16 + i * 4 + k)
                       for j in range(3) for i in range(4) for k in range(4)])
    NB = _sel(shapes, [(_tcol(j, k), j * 16 + i * 4 + k)
                       for j in range(3) for i in range(4) for k in range(4)])
    NC = np.zeros((48, 16), np.float32)
    for j in range(3):
        for i in range(4):
            for k in range(4):
                NC[j * 16 + i * 4 + k, i * 4 + k] = 1.0
    sh2 = {"tab": (TW, 24), "fc": (MF, 24)}
    PA = _sel(sh2, [(_tcol(i, j), i * 8 + k * 4 + j)
                    for i in range(3) for k in range(2) for j in range(4)])
    Q = np.zeros((40, 24), np.float32)
    R = np.zeros((24, TW), np.float32)
    for i in range(3):
        for k in range(2):
            for j in range(4):
                Q[2 * j + k, i * 8 + k * 4 + j] = 1.0
                R[i * 8 + k * 4 + j, i * 2 + k] = 1.0
    S = np.zeros((40, TW), np.float32)
    for m in range(SD):
        S[8 + m, 16 + m] = 1.0
    E6 = np.zeros((MF, 1), np.float32)
    E6[6, 0] = 1.0
    return (NA["tab"], NA["fc"], NB["tab"], NB["fc"], NC,
            PA["tab"], PA["fc"], Q, R, S, E6)


_EDGE_C = _edge_consts()
_NODE_C = _node_consts()


def _silu(x):
    return x / (1.0 + jnp.exp(-x))


def _mm(a, b):
    return jnp.dot(a, b, preferred_element_type=jnp.float32)


def _edge_body(gs_ref, gd_ref, ef_ref, es_ref,
               w1g, w1gs, w1gd, w1es, b1, w2, b2, w3, b3,
               ea_gs, ea_gd, ea_ef, eb_gs, eb_gd, eb_ef, ec,
               ma_gs, ma_gd, ma_ef, mb, mc, ms, cnt,
               msgf_ref, msgs_ref):
    gs = gs_ref[...]
    gd = gd_ref[...]
    ef = ef_ref[...]
    ag = _mm(gs, ea_gs[...]) + _mm(gd, ea_gd[...]) + _mm(ef, ea_ef[...])
    bg = _mm(gs, eb_gs[...]) + _mm(gd, eb_gd[...]) + _mm(ef, eb_ef[...])
    gram = _mm(ag * bg, ec[...])
    ss = jnp.sum(gram * gram, axis=1, keepdims=True)
    invn = 1.0 / jnp.maximum(jnp.sqrt(ss), 1e-12)
    h = _silu(_mm(gram, w1g[...]) * invn + _mm(gs, w1gs[...])
              + _mm(gd, w1gd[...]) + _mm(es_ref[...], w1es[...]) + b1[...])
    h = _silu(_mm(h, w2[...]) + b2[...])
    cc = _mm(h, w3[...]) + b3[...]
    am = _mm(gs, ma_gs[...]) + _mm(gd, ma_gd[...]) + _mm(ef, ma_ef[...])
    bm = _mm(cc, mb[...])
    msgf_ref[...] = _mm(am * bm, mc[...]) + cnt[...]
    msgs_ref[...] = _mm(cc, ms[...])


def _edge_call(gsrc, gdst, efp, esp, nw, ecst):
    nb = EP_ // BE
    full = lambda a: pl.BlockSpec(a.shape, lambda i: (0,) * a.ndim)
    return pl.pallas_call(
        _edge_body,
        grid=(nb,),
        in_specs=[
            pl.BlockSpec((BE, TW), lambda i: (i, 0)),
            pl.BlockSpec((BE, TW), lambda i: (i, 0)),
            pl.BlockSpec((BE, 3), lambda i: (i, 0)),
            pl.BlockSpec((BE, 4), lambda i: (i, 0)),
        ] + [full(a) for a in nw] + [full(a) for a in ecst],
        out_specs=[
            pl.BlockSpec((BE, MF), lambda i: (i, 0)),
            pl.BlockSpec((BE, SD), lambda i: (i, 0)),
        ],
        out_shape=[
            jax.ShapeDtypeStruct((EP_, MF), jnp.float32),
            jax.ShapeDtypeStruct((EP_, SD), jnp.float32),
        ],
    )(gsrc, gdst, efp, esp, *nw, *ecst)


def _node_body(tab_ref, fp0, fp1, sp0, sp1,
               w1g, w1tab, w1sc, b1, w2, b2, w3, b3,
               na_tab, na_fc, nb_tab, nb_fc, nc,
               pa_tab, pa_fc, q, r, s_, e6, out_ref):
    tab = tab_ref[...]
    fsum = fp0[...] + fp1[...]
    inv = 1.0 / jnp.maximum(_mm(fsum, e6[...]), 1.0)
    fci = fsum * inv
    ssum = (sp0[...] + sp1[...]) * inv
    ag = _mm(tab, na_tab[...]) + _mm(fci, na_fc[...])
    bg = _mm(tab, nb_tab[...]) + _mm(fci, nb_fc[...])
    gram = _mm(ag * bg, nc[...])
    ss = jnp.sum(gram * gram, axis=1, keepdims=True)
    invn = 1.0 / jnp.maximum(jnp.sqrt(ss), 1e-12)
    h = _silu(_mm(gram, w1g[...]) * invn + _mm(tab, w1tab[...])
              + _mm(ssum, w1sc[...]) + b1[...])
    h = _silu(_mm(h, w2[...]) + b2[...])
    tc = _mm(h, w3[...]) + b3[...]
    a2 = _mm(tab, pa_tab[...]) + _mm(fci, pa_fc[...])
    b2_ = _mm(tc, q[...])
    out_ref[...] = _mm(a2 * b2_, r[...]) + _mm(tc, s_[...])


def _node_call(tab, fpart, spart, sw, ncst):
    nb = NP_ // BN
    off = NP_ // BN
    full = lambda a: pl.BlockSpec(a.shape, lambda i: (0,) * a.ndim)
    return pl.pallas_call(
        _node_body,
        grid=(nb,),
        in_specs=[
            pl.BlockSpec((BN, TW), lambda i: (i, 0)),
            pl.BlockSpec((BN, MF), lambda i: (i, 0)),
            pl.BlockSpec((BN, MF), lambda i: (i + off, 0)),
            pl.BlockSpec((BN, SD), lambda i: (i, 0)),
            pl.BlockSpec((BN, SD), lambda i: (i + off, 0)),
        ] + [full(a) for a in sw] + [full(a) for a in ncst],
        out_specs=pl.BlockSpec((BN, TW), lambda i: (i, 0)),
        out_shape=jax.ShapeDtypeStruct((NP_, TW), jnp.float32),
    )(tab, fpart, fpart, spart, spart, *sw, *ncst)


def _edge_weights(p):
    w1 = p["W1"]
    return (w1[:25],
            jnp.pad(w1[25:57], ((16, 0), (0, 0))),
            jnp.pad(w1[57:89], ((16, 0), (0, 0))),
            w1[89:93],
            p["b1"].reshape(1, -1), p["W2"], p["b2"].reshape(1, -1),
            p["W3"], p["b3"].reshape(1, -1))


def _node_weights(p):
    w1 = p["W1"]
    return (w1[:16],
            jnp.pad(w1[16:48], ((16, 0), (0, 0))),
            w1[48:80],
            p["b1"].reshape(1, -1), p["W2"], p["b2"].reshape(1, -1),
            p["W3"], p["b3"].reshape(1, -1))


def kernel(f, s, edge_index, edge_f, edge_s, net, self_net):
    ei = edge_index.astype(jnp.int32)
    pad = jnp.full((EP_ - E,), N, jnp.int32)
    srcp = jnp.concatenate([ei[0], pad])
    dstp = jnp.concatenate([ei[1], pad])
    efp = jnp.pad(edge_f.reshape(E, 3), ((0, EP_ - E), (0, 0)))
    esp = jnp.pad(edge_s, ((0, EP_ - E), (0, 0)))
    tab = jnp.concatenate([
        jnp.pad(f.reshape(N, 6), ((0, NP_ - N), (0, 0))),
        jnp.zeros((NP_, 10), jnp.float32),
        jnp.pad(s, ((0, NP_ - N), (0, 0))),
    ], axis=1)
    zf = jnp.zeros((NP_, MF), jnp.float32)
    zs = jnp.zeros((NP_, SD), jnp.float32)
    nw = _edge_weights(net)
    sw = _node_weights(self_net)
    ecst = tuple(jnp.asarray(m) for m in _EDGE_C)
    ncst = tuple(jnp.asarray(m) for m in _NODE_C)
    gather_k = _build_gather()
    scatter_f = _build_scatter(MF)
    scatter_s = _build_scatter(SD)
    for _ in range(PSTEP):
        gsrc, gdst = gather_k(tab, srcp, dstp)
        msgf, msgs = _edge_call(gsrc, gdst, efp, esp, nw, ecst)
        fpart = scatter_f(msgf, srcp, zf)
        spart = scatter_s(msgs, srcp, zs)
        tab = _node_call(tab, fpart, spart, sw, ncst)
    return tab[:N, :6].reshape(N, 3, FD), tab[:N, 16:48]
